# speedup vs baseline: 184.1826x; 184.1826x over previous
import jax
import jax.numpy as jnp
from jax import lax
from jax.experimental import pallas as pl
from jax.experimental.pallas import tpu as pltpu
from jax.experimental.pallas import tpu_sc as plsc

_V = 65536
_NNZ = 4294967
_ALPHA = 0.1

_W = 2048
_NWIN = _NNZ // _W
_MAIN = _NWIN * _W
_TA_BASE = _MAIN
_TA_LEN = ((_NNZ - _MAIN) // 16) * 16
_TB_BASE = _TA_BASE + _TA_LEN
_TB_LEN = _NNZ - _TB_BASE

_NC = 2
_NS = 16
_NW = _NC * _NS
_WINS_PER_W = -(-_NWIN // _NW)
_SLICE = _V // _NS


def _mul_loop(n, a_ref, b_ref, o_ref):
  @pl.loop(0, n, step=16)
  def _(i):
    o_ref[pl.ds(i, 16)] = a_ref[pl.ds(i, 16)] * b_ref[pl.ds(i, 16)]


def _sc_body(logits_hbm, rows_hbm, cols_hbm, vals_hbm, part_hbm,
             rows_v, cols_v, vals_v, gath_v, prod_v,
             rows_s, cols_s, vals_s, gath_s, prod_s,
             idx16, r16, c16, v16, g16, p16,
             stage_v, logits_sh, acc_sh):
  cid = lax.axis_index("c")
  sid = lax.axis_index("s")
  wid = sid * _NC + cid

  @pl.loop(0, _SLICE, step=16)
  def _(i):
    stage_v[pl.ds(i, 16)] = jnp.zeros((16,), jnp.float32)

  pltpu.sync_copy(stage_v, acc_sh.at[pl.ds(sid * _SLICE, _SLICE)])
  pltpu.sync_copy(logits_hbm.at[pl.ds(sid * _SLICE, _SLICE)],
                  logits_sh.at[pl.ds(sid * _SLICE, _SLICE)])
  plsc.subcore_barrier()

  @pl.loop(0, _WINS_PER_W)
  def _(k):
    g = wid + k * _NW

    @pl.when(g < _NWIN)
    def _():
      base = g * _W
      pltpu.sync_copy(rows_hbm.at[pl.ds(base, _W)], rows_v)
      pltpu.sync_copy(cols_hbm.at[pl.ds(base, _W)], cols_v)
      pltpu.sync_copy(vals_hbm.at[pl.ds(base, _W)], vals_v)
      pltpu.sync_copy(logits_sh.at[cols_v], gath_v)
      _mul_loop(_W, gath_v, vals_v, prod_v)
      pltpu.sync_copy(prod_v, acc_sh.at[rows_v], add=True)

  @pl.when(wid == _NW - 1)
  def _():
    pltpu.sync_copy(rows_hbm.at[pl.ds(_TA_BASE, _TA_LEN)], rows_s)
    pltpu.sync_copy(cols_hbm.at[pl.ds(_TA_BASE, _TA_LEN)], cols_s)
    pltpu.sync_copy(vals_hbm.at[pl.ds(_TA_BASE, _TA_LEN)], vals_s)
    pltpu.sync_copy(logits_sh.at[cols_s], gath_s)
    _mul_loop(_TA_LEN, gath_s, vals_s, prod_s)
    pltpu.sync_copy(prod_s, acc_sh.at[rows_s], add=True)

  @pl.when(wid == _NW - 2)
  def _():
    lane = lax.iota(jnp.int32, 16)
    idx16[...] = jnp.minimum(lane + _TB_BASE, _NNZ - 1)
    pltpu.sync_copy(rows_hbm.at[idx16], r16)
    pltpu.sync_copy(cols_hbm.at[idx16], c16)
    pltpu.sync_copy(vals_hbm.at[idx16], v16)
    v16[...] = jnp.where(lane < _TB_LEN, v16[...], 0.0)
    pltpu.sync_copy(logits_sh.at[c16], g16)
    p16[...] = g16[...] * v16[...]
    pltpu.sync_copy(p16, acc_sh.at[r16], add=True)

  plsc.subcore_barrier()
  pltpu.sync_copy(acc_sh.at[pl.ds(sid * _SLICE, _SLICE)],
                  part_hbm.at[cid, pl.ds(sid * _SLICE, _SLICE)])


def _sc_spmv(logits, rows, cols, vals):
  kern = pl.kernel(
      _sc_body,
      out_type=jax.ShapeDtypeStruct((_NC, _V), jnp.float32),
      mesh=plsc.VectorSubcoreMesh(core_axis_name="c", subcore_axis_name="s"),
      scratch_types=[
          pltpu.VMEM((_W,), jnp.int32),
          pltpu.VMEM((_W,), jnp.int32),
          pltpu.VMEM((_W,), jnp.float32),
          pltpu.VMEM((_W,), jnp.float32),
          pltpu.VMEM((_W,), jnp.float32),
          pltpu.VMEM((_TA_LEN,), jnp.int32),
          pltpu.VMEM((_TA_LEN,), jnp.int32),
          pltpu.VMEM((_TA_LEN,), jnp.float32),
          pltpu.VMEM((_TA_LEN,), jnp.float32),
          pltpu.VMEM((_TA_LEN,), jnp.float32),
          pltpu.VMEM((16,), jnp.int32),
          pltpu.VMEM((16,), jnp.int32),
          pltpu.VMEM((16,), jnp.int32),
          pltpu.VMEM((16,), jnp.float32),
          pltpu.VMEM((16,), jnp.float32),
          pltpu.VMEM((16,), jnp.float32),
          pltpu.VMEM((_SLICE,), jnp.float32),
          pltpu.VMEM_SHARED((_V,), jnp.float32),
          pltpu.VMEM_SHARED((_V,), jnp.float32),
      ],
  )
  return kern(logits, rows, cols, vals)


def _combine_body(p_ref, l_ref, o_ref):
  o_ref[...] = p_ref[0] + p_ref[1] - _ALPHA * l_ref[...]


def kernel(logits, S_rows, S_cols, S_vals):
  parts = _sc_spmv(logits, S_rows, S_cols, S_vals)
  out = pl.pallas_call(
      _combine_body,
      out_shape=jax.ShapeDtypeStruct((_V,), jnp.float32),
  )(parts, logits)
  return out


# speedup vs baseline: 313.6040x; 1.7027x over previous
import dataclasses

import jax
import jax.numpy as jnp
from jax import lax
from jax.experimental import pallas as pl
from jax.experimental.pallas import tpu as pltpu
from jax.experimental.pallas import tpu_sc as plsc

_V = 65536
_NNZ = 4294967
_ALPHA = 0.1

_W = 2048
_NWIN = _NNZ // _W
_MAIN = _NWIN * _W
_TA_BASE = _MAIN
_TA_LEN = ((_NNZ - _MAIN) // 16) * 16
_TB_BASE = _TA_BASE + _TA_LEN
_TB_LEN = _NNZ - _TB_BASE

_NC = 2
_NS = 16
_NW = _NC * _NS
_WINS_PER_W = -(-_NWIN // _NW)
_HI_MASK = -65536


def _issue_window(rows_hbm, cols_hbm, vals_hbm, rb, cb, vb, sem, base):
  pltpu.async_copy(rows_hbm.at[pl.ds(base, _W)], rb, sem)
  pltpu.async_copy(cols_hbm.at[pl.ds(base, _W)], cb, sem)
  pltpu.async_copy(vals_hbm.at[pl.ds(base, _W)], vb, sem)


def _wait_window(rows_hbm, cols_hbm, vals_hbm, rb, cb, vb, sem, base):
  pltpu.make_async_copy(rows_hbm.at[pl.ds(base, _W)], rb, sem).wait()
  pltpu.make_async_copy(cols_hbm.at[pl.ds(base, _W)], cb, sem).wait()
  pltpu.make_async_copy(vals_hbm.at[pl.ds(base, _W)], vb, sem).wait()


def _spmv_vregs(n, rows_ref, cols_ref, vals_ref, packed_ref, acc_ref):
  @pl.loop(0, n, step=16)
  def _(i):
    cols = cols_ref[pl.ds(i, 16)]
    rows = rows_ref[pl.ds(i, 16)]
    vals = vals_ref[pl.ds(i, 16)]
    word = plsc.load_gather(packed_ref, [lax.shift_right_logical(cols, 1)])
    hi = lax.bitwise_and(word, jnp.int32(_HI_MASK))
    lo = lax.shift_left(word, 16)
    g32 = jnp.where(lax.bitwise_and(cols, 1) == 1, hi, lo)
    prod = plsc.bitcast(g32, jnp.float32) * vals
    plsc.addupdate_scatter(acc_ref, [rows], prod)


def _sc_body(packed_hbm, rows_hbm, cols_hbm, vals_hbm, part_hbm,
             packed_v, acc_v,
             rows0, cols0, vals0, rows1, cols1, vals1,
             rows_s, cols_s, vals_s,
             idx16, r16, c16, v16,
             sem0, sem1):
  cid = lax.axis_index("c")
  sid = lax.axis_index("s")
  wid = sid * _NC + cid

  pltpu.async_copy(packed_hbm, packed_v, sem0)

  @pl.loop(0, _V, step=64)
  def _(i):
    acc_v[pl.ds(i, 16)] = jnp.zeros((16,), jnp.float32)
    acc_v[pl.ds(i + 16, 16)] = jnp.zeros((16,), jnp.float32)
    acc_v[pl.ds(i + 32, 16)] = jnp.zeros((16,), jnp.float32)
    acc_v[pl.ds(i + 48, 16)] = jnp.zeros((16,), jnp.float32)

  pltpu.make_async_copy(packed_hbm, packed_v, sem0).wait()

  bufs = ((rows0, cols0, vals0, sem0), (rows1, cols1, vals1, sem1))

  def win_base(k):
    return jnp.minimum(wid + k * _NW, _NWIN - 1) * _W

  for b in (0, 1):
    rb, cb, vb, sem = bufs[b]
    _issue_window(rows_hbm, cols_hbm, vals_hbm, rb, cb, vb, sem, win_base(b))

  @pl.loop(0, _WINS_PER_W, step=2)
  def _(k):
    for b in (0, 1):
      kk = k + b
      rb, cb, vb, sem = bufs[b]
      _wait_window(rows_hbm, cols_hbm, vals_hbm, rb, cb, vb, sem,
                   win_base(kk))

      @pl.when(wid + kk * _NW < _NWIN)
      def _():
        _spmv_vregs(_W, rb, cb, vb, packed_v, acc_v)

      @pl.when(kk + 2 < _WINS_PER_W)
      def _():
        _issue_window(rows_hbm, cols_hbm, vals_hbm, rb, cb, vb, sem,
                      win_base(kk + 2))

  @pl.when(wid == _NW - 1)
  def _():
    pltpu.sync_copy(rows_hbm.at[pl.ds(_TA_BASE, _TA_LEN)], rows_s)
    pltpu.sync_copy(cols_hbm.at[pl.ds(_TA_BASE, _TA_LEN)], cols_s)
    pltpu.sync_copy(vals_hbm.at[pl.ds(_TA_BASE, _TA_LEN)], vals_s)
    _spmv_vregs(_TA_LEN, rows_s, cols_s, vals_s, packed_v, acc_v)

  @pl.when(wid == _NW - 2)
  def _():
    lane = lax.iota(jnp.int32, 16)
    idx16[...] = jnp.minimum(lane + _TB_BASE, _NNZ - 1)
    pltpu.sync_copy(rows_hbm.at[idx16], r16)
    pltpu.sync_copy(cols_hbm.at[idx16], c16)
    pltpu.sync_copy(vals_hbm.at[idx16], v16)
    v16[...] = jnp.where(lane < _TB_LEN, v16[...], 0.0)
    _spmv_vregs(16, r16, c16, v16, packed_v, acc_v)

  pltpu.sync_copy(acc_v, part_hbm.at[cid, sid])


def _sc_compiler_params():
  cp = pltpu.CompilerParams()
  if "needs_layout_passes" in pltpu.CompilerParams.__dataclass_fields__:
    cp = dataclasses.replace(cp, needs_layout_passes=False)
  return cp


def _sc_spmv(packed, rows, cols, vals):
  kern = pl.kernel(
      _sc_body,
      out_type=jax.ShapeDtypeStruct((_NC, _NS, _V), jnp.float32),
      mesh=plsc.VectorSubcoreMesh(core_axis_name="c", subcore_axis_name="s"),
      compiler_params=_sc_compiler_params(),
      scratch_types=[
          pltpu.VMEM((_V // 2,), jnp.int32),
          pltpu.VMEM((_V,), jnp.float32),
          pltpu.VMEM((_W,), jnp.int32),
          pltpu.VMEM((_W,), jnp.int32),
          pltpu.VMEM((_W,), jnp.float32),
          pltpu.VMEM((_W,), jnp.int32),
          pltpu.VMEM((_W,), jnp.int32),
          pltpu.VMEM((_W,), jnp.float32),
          pltpu.VMEM((_TA_LEN,), jnp.int32),
          pltpu.VMEM((_TA_LEN,), jnp.int32),
          pltpu.VMEM((_TA_LEN,), jnp.float32),
          pltpu.VMEM((16,), jnp.int32),
          pltpu.VMEM((16,), jnp.int32),
          pltpu.VMEM((16,), jnp.int32),
          pltpu.VMEM((16,), jnp.float32),
          pltpu.SemaphoreType.DMA,
          pltpu.SemaphoreType.DMA,
      ],
  )
  return kern(packed, rows, cols, vals)


def _combine_body(p_ref, l_ref, o_ref):
  o_ref[...] = jnp.sum(p_ref[...], axis=0) - _ALPHA * l_ref[...]


@jax.jit
def kernel(logits, S_rows, S_cols, S_vals):
  packed = lax.bitcast_convert_type(
      logits.astype(jnp.bfloat16).reshape(_V // 2, 2), jnp.int32)
  parts = _sc_spmv(packed, S_rows, S_cols, S_vals)
  out = pl.pallas_call(
      _combine_body,
      out_shape=jax.ShapeDtypeStruct((_V,), jnp.float32),
  )(parts.reshape(_NC * _NS, _V), logits)
  return out
